# Initial kernel scaffold; baseline (speedup 1.0000x reference)
import jax
import jax.numpy as jnp
from jax.experimental import pallas as pl


def kernel(pn_feat, obj_pos, obj_rot, obj_size, obj_attr, edge_index, edge_type, mlp_w1, mlp_b1, mlp_w2, mlp_b2, mlp_w3, mlp_b3, w1, root1, b1, w2, root2, b2, w3, root3, b3):
    raise NotImplementedError("write your pallas kernel here")


# speedup vs baseline: 4.5092x; 4.5092x over previous
import functools

import jax
import jax.numpy as jnp
from jax import lax
from jax.experimental import pallas as pl
from jax.experimental.pallas import tpu as pltpu
from jax.experimental.pallas import tpu_sc as plsc

N = 10000
E = 640000
NREL = 6
HID = 64
NP = 10240
BLK = 512
NB = NP // BLK
QN = NP // 4
QB = QN // BLK
TR = (NREL + 1) * NP
RVAL = 3 * QN
RACC = RVAL + 128
TRASH = RVAL
NHALF = NP // 2
CACC = NHALF + 128
CTRASH = NHALF
NTILE = 16
CK = 128
NCHUNK = 320
NCH2 = NCHUNK + 2
EPAD = NTILE * NCHUNK * CK
ZR = RACC // NTILE
WB = RVAL // NTILE
ZRC = CACC // NTILE
WBC = NHALF // NTILE


def _prep_body(pn, scene, w1, b1, w2, b2, w3, b3, out):
    h = jnp.maximum(jnp.dot(scene[...], w1[...], preferred_element_type=jnp.float32) + b1[...], 0.0)
    h = jnp.maximum(jnp.dot(h, w2[...], preferred_element_type=jnp.float32) + b2[...], 0.0)
    emb = jnp.dot(h, w3[...], preferred_element_type=jnp.float32) + b3[...]
    out[...] = jnp.concatenate(
        [pn[...], emb, jnp.zeros((BLK, 7), jnp.float32)], axis=1)


def _prep(pn_pad, scene_pad, w1, b1, w2, b2, w3, b3):
    cmap = lambda i: (0, 0)
    return pl.pallas_call(
        _prep_body,
        grid=(NB,),
        in_specs=[
            pl.BlockSpec((BLK, 57), lambda i: (i, 0)),
            pl.BlockSpec((BLK, 22), lambda i: (i, 0)),
            pl.BlockSpec((22, 32), cmap),
            pl.BlockSpec((1, 32), cmap),
            pl.BlockSpec((32, 32), cmap),
            pl.BlockSpec((1, 32), cmap),
            pl.BlockSpec((32, 64), cmap),
            pl.BlockSpec((1, 64), cmap),
        ],
        out_specs=pl.BlockSpec((BLK, 128), lambda i: (i, 0)),
        out_shape=jax.ShapeDtypeStruct((NP, 128), jnp.float32),
    )(pn_pad, scene_pad, w1, b1, w2, b2, w3, b3)


def _htable_body(x, w, out):
    r = pl.program_id(0)
    h = jnp.dot(x[...], w[0], preferred_element_type=jnp.float32)
    z = jnp.zeros((BLK, HID), jnp.float32)
    out[...] = jnp.where(lax.rem(r, 2) == 0,
                         jnp.concatenate([h, z], axis=1),
                         jnp.concatenate([z, h], axis=1))


def _htable(x, w_all, k):
    return pl.pallas_call(
        _htable_body,
        grid=(NREL + 1, NB),
        in_specs=[
            pl.BlockSpec((BLK, k), lambda r, i: (i, 0)),
            pl.BlockSpec((1, k, HID), lambda r, i: (r, 0, 0)),
        ],
        out_specs=pl.BlockSpec((BLK, 128), lambda r, i: (r * NB + i, 0)),
        out_shape=jax.ShapeDtypeStruct((TR, 128), jnp.float32),
    )(x, w_all)


def _edge_pipeline(src_ref, idx_ref, acc, ib_a, ib_b, rba, rbb,
                   sga, sgb, sia, sib):
    pltpu.async_copy(idx_ref.at[0], ib_a, sia)
    pltpu.async_copy(idx_ref.at[1], ib_b, sib)
    pltpu.make_async_copy(idx_ref.at[0], ib_a, sia).wait()
    pltpu.async_copy(src_ref.at[ib_a.at[0]], rba, sga)

    def body(jj, _):
        j = jj * 2
        pltpu.make_async_copy(src_ref.at[ib_a.at[0]], rba, sga).wait()
        pltpu.make_async_copy(idx_ref.at[0], ib_b, sib).wait()
        pltpu.async_copy(src_ref.at[ib_b.at[0]], rbb, sgb)
        pltpu.sync_copy(rba, acc.at[ib_a.at[1]], add=True)
        pltpu.async_copy(idx_ref.at[j + 2], ib_a, sia)
        pltpu.make_async_copy(src_ref.at[ib_b.at[0]], rbb, sgb).wait()
        pltpu.make_async_copy(idx_ref.at[0], ib_a, sia).wait()
        pltpu.async_copy(src_ref.at[ib_a.at[0]], rba, sga)
        pltpu.sync_copy(rbb, acc.at[ib_b.at[1]], add=True)
        pltpu.async_copy(idx_ref.at[j + 3], ib_b, sib)
        return 0

    lax.fori_loop(0, NCHUNK // 2, body, 0, unroll=False)
    pltpu.make_async_copy(src_ref.at[ib_a.at[0]], rba, sga).wait()
    pltpu.make_async_copy(idx_ref.at[0], ib_b, sib).wait()


def _sc_edges_body(tbl, idxm, zrows, accout,
                   ib_a, ib_b, rba, rbb, acc, sga, sgb, sia, sib):
    c = lax.axis_index("c")
    s = lax.axis_index("s")
    pltpu.sync_copy(zrows, acc.at[pl.ds(s * ZR, ZR)])
    plsc.subcore_barrier()
    _edge_pipeline(tbl, idxm.at[c, s], acc, ib_a, ib_b, rba, rbb,
                   sga, sgb, sia, sib)
    plsc.subcore_barrier()
    pltpu.sync_copy(acc.at[pl.ds(s * WB, WB)],
                    accout.at[pl.ds(c * RVAL + s * WB, WB)])


def _sc_edges(tbl, idxm_p, zrows):
    mesh = plsc.VectorSubcoreMesh(core_axis_name="c", subcore_axis_name="s",
                                  num_cores=2, num_subcores=NTILE)
    kfn = pl.kernel(
        _sc_edges_body,
        out_type=jax.ShapeDtypeStruct((2 * RVAL, 128), jnp.float32),
        mesh=mesh,
        scratch_types=[
            pltpu.VMEM((2, CK), jnp.int32),
            pltpu.VMEM((2, CK), jnp.int32),
            pltpu.VMEM((CK, 128), jnp.float32),
            pltpu.VMEM((CK, 128), jnp.float32),
            pltpu.VMEM_SHARED((RACC, 128), jnp.float32),
            pltpu.SemaphoreType.DMA,
            pltpu.SemaphoreType.DMA,
            pltpu.SemaphoreType.DMA,
            pltpu.SemaphoreType.DMA,
        ],
    )
    return kfn(tbl, idxm_p, zrows)


def _sc_counts_body(parityt, idxc, zrows, cout,
                    ib_a, ib_b, rba, rbb, ones_s, acc, sga, sgb, sia, sib):
    c = lax.axis_index("c")
    s = lax.axis_index("s")
    pltpu.sync_copy(zrows, acc.at[pl.ds(s * ZRC, ZRC)])

    @pl.when(s == 0)
    def _():
        pltpu.sync_copy(parityt, ones_s)

    plsc.subcore_barrier()
    _edge_pipeline(ones_s, idxc.at[c, s], acc, ib_a, ib_b, rba, rbb,
                   sga, sgb, sia, sib)
    plsc.subcore_barrier()
    pltpu.sync_copy(acc.at[pl.ds(s * WBC, WBC)],
                    cout.at[pl.ds(c * NHALF + s * WBC, WBC)])


def _sc_counts(parityt, idxc, zrowsc):
    mesh = plsc.VectorSubcoreMesh(core_axis_name="c", subcore_axis_name="s",
                                  num_cores=2, num_subcores=NTILE)
    kfn = pl.kernel(
        _sc_counts_body,
        out_type=jax.ShapeDtypeStruct((NP, 128), jnp.float32),
        mesh=mesh,
        scratch_types=[
            pltpu.VMEM((2, CK), jnp.int32),
            pltpu.VMEM((2, CK), jnp.int32),
            pltpu.VMEM((CK, 128), jnp.float32),
            pltpu.VMEM((CK, 128), jnp.float32),
            pltpu.VMEM_SHARED((16, 128), jnp.float32),
            pltpu.VMEM_SHARED((CACC, 128), jnp.float32),
            pltpu.SemaphoreType.DMA,
            pltpu.SemaphoreType.DMA,
            pltpu.SemaphoreType.DMA,
            pltpu.SemaphoreType.DMA,
        ],
    )
    return kfn(parityt, idxc, zrowsc)


def _combine_body(accb, cb, rt, b, out, *, act):
    r = pl.program_id(1)

    @pl.when(r == 0)
    def _():
        out[...] = rt[:, :HID] + b[...]

    col = lax.broadcasted_iota(jnp.int32, (1, 128), 1)
    cnt = jnp.sum(jnp.where(col == 8 * r, cb[...], 0.0), axis=1,
                  keepdims=True)
    inv = 1.0 / jnp.maximum(cnt, 1.0)
    odd = lax.rem(r, 2) == 1
    acch = jnp.where(odd, accb[:, HID:], accb[:, :HID])
    out[...] += acch * inv

    if act:
        @pl.when(r == NREL - 1)
        def _():
            out[...] = jnp.maximum(out[...], 0.0)


def _acc_map(i, r):
    q, im = lax.div(i, QB), lax.rem(i, QB)
    return (q * (3 * QB) + lax.div(r, 2) * QB + im, 0)


def _combine(acc2, cnts, ht, bias, act):
    return pl.pallas_call(
        functools.partial(_combine_body, act=act),
        grid=(NB, NREL),
        in_specs=[
            pl.BlockSpec((BLK, 128), _acc_map),
            pl.BlockSpec((BLK, 128), lambda i, r: (i, 0)),
            pl.BlockSpec((BLK, 128), lambda i, r: (NREL * NB + i, 0)),
            pl.BlockSpec((1, HID), lambda i, r: (0, 0)),
        ],
        out_specs=pl.BlockSpec((BLK, HID), lambda i, r: (i, 0)),
        out_shape=jax.ShapeDtypeStruct((NP, HID), jnp.float32),
    )(acc2, cnts, ht, bias)


def kernel(pn_feat, obj_pos, obj_rot, obj_size, obj_attr, edge_index,
           edge_type, mlp_w1, mlp_b1, mlp_w2, mlp_b2, mlp_w3, mlp_b3,
           w1, root1, b1, w2, root2, b2, w3, root3, b3):
    f32 = jnp.float32
    src = edge_index[0].astype(jnp.int32)
    dst = edge_index[1].astype(jnp.int32)
    et = edge_type.astype(jnp.int32)

    pad = EPAD - E
    gidx = jnp.concatenate([et * NP + src, jnp.zeros((pad,), jnp.int32)])
    gidx = gidx.reshape(NTILE, NCHUNK, CK)
    cidx = jnp.concatenate([et, jnp.full((pad,), 8, jnp.int32)])
    cidx = cidx.reshape(NTILE, NCHUNK, CK)

    def pack(g, s_, dg, ds_):
        m = jnp.stack([g, s_], axis=2)
        d = jnp.stack([dg, ds_], axis=2)
        return jnp.concatenate([m, d], axis=1)

    grp = (et // 2) * QN
    quarters = []
    for q in range(4):
        own = (dst // QN) == q
        sq = jnp.where(own, grp + dst - q * QN, TRASH)
        sq = jnp.concatenate([sq, jnp.full((pad,), TRASH, jnp.int32)])
        quarters.append(sq.reshape(NTILE, NCHUNK, CK))
    dum_g = jnp.zeros((NTILE, 2, CK), jnp.int32)
    dum_s = jnp.full((NTILE, 2, CK), TRASH, jnp.int32)
    idxm = jnp.stack([
        jnp.stack([pack(gidx, quarters[0], dum_g, dum_s),
                   pack(gidx, quarters[1], dum_g, dum_s)]),
        jnp.stack([pack(gidx, quarters[2], dum_g, dum_s),
                   pack(gidx, quarters[3], dum_g, dum_s)]),
    ])

    halves = []
    for c in range(2):
        own = (dst // NHALF) == c
        sq = jnp.where(own, dst - c * NHALF, CTRASH)
        sq = jnp.concatenate([sq, jnp.full((pad,), CTRASH, jnp.int32)])
        halves.append(sq.reshape(NTILE, NCHUNK, CK))
    dum_cg = jnp.full((NTILE, 2, CK), 8, jnp.int32)
    dum_cs = jnp.full((NTILE, 2, CK), CTRASH, jnp.int32)
    idxc = jnp.stack([pack(cidx, halves[0], dum_cg, dum_cs),
                      pack(cidx, halves[1], dum_cg, dum_cs)])

    zrows = jnp.zeros((ZR, 128), f32)
    zrowsc = jnp.zeros((ZRC, 128), f32)
    rr = jnp.arange(16)[:, None]
    cc = jnp.arange(128)[None, :] // 8
    parityt = ((rr == cc) & (rr < NREL)).astype(f32)

    scene = jnp.concatenate([obj_pos, obj_rot, obj_size, obj_attr], axis=1)
    scene_pad = jnp.pad(scene, ((0, NP - N), (0, 0)))
    pn_pad = jnp.pad(pn_feat, ((0, NP - N), (0, 0)))

    w1p = jnp.concatenate([jnp.pad(w1, ((0, 0), (0, 7), (0, 0))),
                           jnp.pad(root1, ((0, 7), (0, 0)))[None]], axis=0)
    w2p = jnp.concatenate([w2, root2[None]], axis=0)
    w3p = jnp.concatenate([w3, root3[None]], axis=0)

    x = _prep(pn_pad, scene_pad, mlp_w1, mlp_b1.reshape(1, -1),
              mlp_w2, mlp_b2.reshape(1, -1), mlp_w3, mlp_b3.reshape(1, -1))

    cnts = _sc_counts(parityt, idxc, zrowsc)

    for wp, bias, k, act in ((w1p, b1, 128, True),
                             (w2p, b2, HID, True),
                             (w3p, b3, HID, False)):
        ht = _htable(x, wp, k)
        a0 = _sc_edges(ht, idxm[0], zrows)
        a1 = _sc_edges(ht, idxm[1], zrows)
        acc2 = jnp.concatenate([a0, a1], axis=0)
        x = _combine(acc2, cnts, ht, bias.reshape(1, -1), act)

    return x[:N][None]
